# baseline (speedup 1.0000x reference)
import functools

import jax
import jax.numpy as jnp
from jax import lax
from jax.experimental import pallas as pl
from jax.experimental.pallas import tpu as pltpu
from jax.experimental.pallas import tpu_sc as plsc

B = 16384
D = 64
NENT = 1000000
NREL = 1000
L = 16
PW = 384
NP = (NENT + PW - 1) // PW
DUMMY = 2 * B
SUBS = 16
CAP = 1024

_info = plsc.get_sparse_core_info()
NC, NS = _info.num_cores, _info.num_subcores
NW = NC * NS
BPW = B // NW
PER = NP // NW
EXTRA = NP - PER * NW

_mesh = plsc.VectorSubcoreMesh(core_axis_name="c", subcore_axis_name="s")
_params = pltpu.CompilerParams(needs_layout_passes=False)


@functools.partial(
    pl.kernel,
    mesh=_mesh,
    out_type=jax.ShapeDtypeStruct((2 * B + 8, 128), jnp.float32),
    compiler_params=_params,
    scratch_types=[
        pltpu.VMEM((2 * B,), jnp.int32),
        pltpu.VMEM((2 * B,), jnp.int32),
        pltpu.VMEM((CAP,), jnp.int32),
        pltpu.VMEM((CAP,), jnp.int32),
        pltpu.VMEM((2, D, PW), jnp.float32),
        pltpu.VMEM((2, 2 * L, 128), jnp.float32),
        pltpu.VMEM((2, 2 * L), jnp.int32),
        pltpu.VMEM((2 * L,), jnp.int32),
        pltpu.VMEM((2 * L,), jnp.int32),
        pltpu.VMEM((1024,), jnp.int32),
        pltpu.VMEM((1024,), jnp.int32),
        pltpu.SMEM((8,), jnp.int32),
        pltpu.SemaphoreType.DMA,
        pltpu.SemaphoreType.DMA,
        pltpu.SemaphoreType.DMA,
    ],
)
def _gather_phase(head_hbm, tail_hbm, entT_hbm, tb_hbm, gout_hbm,
                  ids_v, pos_v, sids_v, spos_v, e_v, stage_v, posb_v,
                  sloc_v, srow_v, ploc_v, prow_v, sm, sem_e0, sem_e1, sem_sc):
    wid = lax.axis_index("s") * NC + lax.axis_index("c")
    np_w = jnp.where(wid < EXTRA, PER + 1, PER)
    p_lo = wid * PER + jnp.minimum(wid, EXTRA)
    id_lo = p_lo * PW
    id_hi = jnp.minimum((p_lo + np_w) * PW, NENT)
    lane = lax.iota(jnp.int32, L)

    sm[0] = jnp.int32(0)
    sm[1] = jnp.int32(0)
    sm[2] = jnp.int32(-1)
    sm[3] = jnp.int32(0)
    sm[4] = jnp.int32(0)

    pltpu.sync_copy(head_hbm, ids_v.at[pl.ds(0, B)])
    pltpu.sync_copy(tail_hbm, ids_v.at[pl.ds(B, B)])

    def prefilter(j, off):
        v = ids_v[pl.ds(j * L, L)]
        m = (v >= id_lo) & (v < id_hi)
        cs = plsc.cumsum(jnp.where(m, 1, 0))
        dest = off + cs - 1
        plsc.store_scatter(ids_v, [dest], v, mask=m)
        plsc.store_scatter(pos_v, [dest], j * L + lane, mask=m)
        return off + cs[15]

    cnt_total = lax.fori_loop(0, 2 * B // L, prefilter, jnp.int32(0))
    nck = (cnt_total + L - 1) // L

    def drain_sc():
        pltpu.make_async_copy(
            stage_v.at[0], gout_hbm.at[posb_v.at[0]], sem_sc).wait()

    def extract2(b, halves):
        ob = sm[1]

        @pl.when(ob >= 2)
        def _():
            drain_sc()

        r = sm[4]
        for h, (lv, rw) in enumerate(halves):
            rowi = lane + h * L
            for c in range(D):
                cf = jnp.full((L,), c, jnp.int32)
                vals = plsc.load_gather(e_v.at[b], [cf, lv])
                plsc.store_scatter(stage_v.at[r], [rowi, cf], vals)
            posb_v[r, pl.ds(h * L, L)] = rw
        pltpu.async_copy(stage_v.at[r], gout_hbm.at[posb_v.at[r]], sem_sc)
        sm[4] = lax.rem(r + 1, 2)
        sm[1] = jnp.where(ob >= 2, ob, ob + 1)

    dummy_lv = jnp.zeros((L,), jnp.int32)
    dummy_rw = jnp.full((L,), DUMMY, jnp.int32)

    def extract(b, lv, rw):
        extract2(b, [(lv, rw), (dummy_lv, dummy_rw)])

    def rescan(lids, lpos, n, rlo, rhi, start, b):
        def chunk(k, carry):
            v = lids[pl.ds(k * L, L)]
            valid = (k * L + lane) < n
            m = valid & (v >= rlo) & (v < rhi)
            any_m = plsc.all_reduce_population_count(m)

            @pl.when(any_m[0] > 0)
            def _():
                pv = lpos[pl.ds(k * L, L)]
                cs = plsc.cumsum(jnp.where(m, 1, 0))
                s0 = sm[0]
                dest = s0 + cs - 1
                plsc.store_scatter(sloc_v, [dest], v - start, mask=m)
                plsc.store_scatter(srow_v, [dest], pv, mask=m)
                scn = s0 + cs[15]

                @pl.when(scn >= L)
                def _():
                    extract(b, sloc_v[pl.ds(0, L)], srow_v[pl.ds(0, L)])
                    sloc_v[pl.ds(0, L)] = sloc_v[pl.ds(L, L)]
                    srow_v[pl.ds(0, L)] = srow_v[pl.ds(L, L)]

                sm[0] = jnp.where(scn >= L, scn - L, scn)
            return carry

        lax.fori_loop(0, (n + L - 1) // L, chunk, jnp.int32(0))

    def compact_sub(s):
        slo = (p_lo + (s * np_w) // SUBS) * PW
        shi = jnp.minimum((p_lo + ((s + 1) * np_w) // SUBS) * PW, NENT)

        def cchunk(k, off):
            v = ids_v[pl.ds(k * L, L)]
            valid = (k * L + lane) < cnt_total
            m = valid & (v >= slo) & (v < shi)
            cs = plsc.cumsum(jnp.where(m, 1, 0))
            dest = jnp.minimum(off + cs - 1, CAP - 1)
            plsc.store_scatter(sids_v, [dest], v, mask=m)
            plsc.store_scatter(spos_v, [dest], pos_v[pl.ds(k * L, L)], mask=m)
            return off + cs[15]

        return lax.fori_loop(0, nck, cchunk, jnp.int32(0))

    def fire_e(p, b, sem):
        pg = p_lo + p

        @pl.when(pg < NP - 1)
        def _():
            start = pl.multiple_of(pg * PW, 128)
            pltpu.async_copy(entT_hbm.at[:, pl.ds(start, PW)], e_v.at[b], sem)

        @pl.when(pg == NP - 1)
        def _():
            for q in range(PW // 128):
                pltpu.async_copy(
                    tb_hbm, e_v.at[b, :, pl.ds(q * 128, 128)], sem)

    def drain_e(sem):
        pltpu.make_async_copy(
            entT_hbm.at[:, pl.ds(0, PW)], e_v.at[0], sem).wait()

    def process(p, b):
        def drp(i, c):
            drain_sc()
            return c

        lax.fori_loop(0, sm[1], drp, jnp.int32(0))
        sm[1] = jnp.int32(0)

        s = (SUBS * (p + 1) - 1) // np_w

        @pl.when(s != sm[2])
        def _():
            sm[3] = compact_sub(s)
            sm[2] = s

        pg = p_lo + p
        rlo = pg * PW
        rhi = jnp.minimum(rlo + PW, NENT)
        start = jnp.where(pg == NP - 1, NENT - 128, rlo)
        csub = sm[3]
        ok = csub <= CAP

        def pchunk(k, off):
            v = sids_v[pl.ds(k * L, L)]
            valid = (k * L + lane) < csub
            m = valid & (v >= rlo) & (v < rhi)
            cs = plsc.cumsum(jnp.where(m, 1, 0))
            dest = jnp.minimum(off + cs - 1, 1024 - 1)
            plsc.store_scatter(ploc_v, [dest], v - start, mask=m)
            plsc.store_scatter(prow_v, [dest], spos_v[pl.ds(k * L, L)], mask=m)
            return off + cs[15]

        pcnt = lax.fori_loop(
            0, jnp.where(ok, (csub + L - 1) // L, 0), pchunk, jnp.int32(0))
        ok2 = ok & (pcnt <= 1024)

        def strip(k, carry):
            halves = []
            for h in range(2):
                j0 = k * 2 * L + h * L
                mvalid = (j0 + lane) < pcnt
                lv = jnp.where(mvalid, ploc_v[pl.ds(j0, L)], 0)
                rw = jnp.where(mvalid, prow_v[pl.ds(j0, L)], DUMMY)
                halves.append((lv, rw))
            extract2(b, halves)
            return carry

        lax.fori_loop(
            0, jnp.where(ok2, (pcnt + 2 * L - 1) // (2 * L), 0),
            strip, jnp.int32(0))

        rescan(ids_v, pos_v, jnp.where(ok, 0, cnt_total), rlo, rhi, start, b)
        rescan(sids_v, spos_v, jnp.where(ok & ~ok2, csub, 0),
               rlo, rhi, start, b)

        scn = sm[0]

        @pl.when(scn > 0)
        def _():
            lv = jnp.where(lane < scn, sloc_v[pl.ds(0, L)], 0)
            rw = jnp.where(lane < scn, srow_v[pl.ds(0, L)], DUMMY)
            extract(b, lv, rw)

        sm[0] = jnp.int32(0)

    fire_e(jnp.int32(0), 0, sem_e0)

    def pair(pp, carry):
        p0 = pp * 2
        p1 = p0 + 1

        @pl.when(p1 < np_w)
        def _():
            fire_e(p1, 1, sem_e1)

        drain_e(sem_e0)
        process(p0, 0)

        @pl.when(p0 + 2 < np_w)
        def _():
            fire_e(p0 + 2, 0, sem_e0)

        @pl.when(p1 < np_w)
        def _():
            drain_e(sem_e1)
            process(p1, 1)

        return carry

    lax.fori_loop(0, (np_w + 1) // 2, pair, jnp.int32(0))

    def dr_final(i, c):
        drain_sc()
        return c

    lax.fori_loop(0, sm[1], dr_final, jnp.int32(0))


CH2 = 64


@functools.partial(
    pl.kernel,
    mesh=_mesh,
    out_type=jax.ShapeDtypeStruct((B,), jnp.float32),
    compiler_params=_params,
    scratch_types=[
        pltpu.VMEM((BPW,), jnp.int32),
        pltpu.VMEM((D, NREL), jnp.float32),
        pltpu.VMEM((2, CH2, 128), jnp.float32),
        pltpu.VMEM((2, CH2, 128), jnp.float32),
        pltpu.VMEM((BPW,), jnp.float32),
        pltpu.SemaphoreType.DMA,
        pltpu.SemaphoreType.DMA,
        pltpu.SemaphoreType.DMA,
    ],
)
def _score_phase(rel_hbm, gout_hbm, relT_hbm, out_hbm,
                 ri_v, rel_v, h_v, t_v, o_v, sem_r, sem0, sem1):
    wid = lax.axis_index("s") * NC + lax.axis_index("c")
    base = wid * BPW
    lane = lax.iota(jnp.int32, L)

    pltpu.sync_copy(rel_hbm.at[pl.ds(base, BPW)], ri_v)
    rel_cp = pltpu.async_copy(relT_hbm, rel_v, sem_r)

    def fire(q, b, sem):
        r0 = base + q * CH2
        pltpu.async_copy(gout_hbm.at[pl.ds(r0, CH2)], h_v.at[b], sem)
        pltpu.async_copy(gout_hbm.at[pl.ds(B + r0, CH2)], t_v.at[b], sem)

    def drain(sem):
        for _ in range(2):
            pltpu.make_async_copy(
                gout_hbm.at[pl.ds(0, CH2)], h_v.at[0], sem).wait()

    def compute(q, b):
        def group(g, carry):
            j0 = g * L
            ridx = ri_v[pl.ds(q * CH2 + j0, L)]
            rows = j0 + lane
            acc = jnp.zeros((L,), jnp.float32)
            for c in range(D):
                cf = jnp.full((L,), c, jnp.int32)
                h = plsc.load_gather(h_v.at[b], [rows, cf])
                t = plsc.load_gather(t_v.at[b], [rows, cf])
                r = plsc.load_gather(rel_v, [cf, ridx])
                acc = acc + jnp.abs(h + r - t)
            o_v[pl.ds(q * CH2 + j0, L)] = acc
            return carry

        lax.fori_loop(0, CH2 // L, group, jnp.int32(0))

    NQ = BPW // CH2
    fire(jnp.int32(0), 0, sem0)
    rel_cp.wait()

    def pairq(qp, carry):
        q0 = qp * 2
        q1 = q0 + 1
        fire(q1, 1, sem1)
        drain(sem0)
        compute(q0, 0)

        @pl.when(q0 + 2 < NQ)
        def _():
            fire(q0 + 2, 0, sem0)

        drain(sem1)
        compute(q1, 1)
        return carry

    lax.fori_loop(0, NQ // 2, pairq, jnp.int32(0))
    pltpu.sync_copy(o_v, out_hbm.at[pl.ds(base, BPW)])


def kernel(head, relation, tail, entity_table, relation_table):
    tail_block = entity_table[NENT - 128:].T
    gout = _gather_phase(head, tail, entity_table.T, tail_block)
    return _score_phase(relation, gout, relation_table.T)


# speedup vs baseline: 2.5017x; 2.5017x over previous
import functools

import jax
import jax.numpy as jnp
from jax import lax
from jax.experimental import pallas as pl
from jax.experimental.pallas import tpu as pltpu
from jax.experimental.pallas import tpu_sc as plsc

B = 16384
D = 64
NENT = 1000000
NREL = 1000
L = 16
PW = 384
NP = (NENT + PW - 1) // PW
DUMMY = 2 * B
SUBS = 16
CAP = 4096

_info = plsc.get_sparse_core_info()
NC, NS = _info.num_cores, _info.num_subcores
NW = NC * NS
BPW = B // NW
PER = NP // NW
EXTRA = NP - PER * NW

_mesh = plsc.VectorSubcoreMesh(core_axis_name="c", subcore_axis_name="s")
_params = pltpu.CompilerParams(needs_layout_passes=False)


@functools.partial(
    pl.kernel,
    mesh=_mesh,
    out_type=jax.ShapeDtypeStruct((2 * B + 8, 128), jnp.float32),
    compiler_params=_params,
    scratch_types=[
        pltpu.VMEM((2 * B,), jnp.int32),
        pltpu.VMEM((2 * B,), jnp.int32),
        pltpu.VMEM((CAP,), jnp.int32),
        pltpu.VMEM((CAP,), jnp.int32),
        pltpu.VMEM((2, D, PW), jnp.float32),
        pltpu.VMEM((2, L, 128), jnp.float32),
        pltpu.VMEM((2, L), jnp.int32),
        pltpu.VMEM((1024,), jnp.int32),
        pltpu.VMEM((1024,), jnp.int32),
        pltpu.SMEM((8,), jnp.int32),
        pltpu.SemaphoreType.DMA,
        pltpu.SemaphoreType.DMA,
        pltpu.SemaphoreType.DMA,
    ],
)
def _gather_phase(head_hbm, tail_hbm, entT_hbm, tb_hbm, gout_hbm,
                  ids_v, pos_v, sids_v, spos_v, e_v, stage_v, posb_v,
                  ploc_v, prow_v, sm, sem_e0, sem_e1, sem_sc):
    wid = lax.axis_index("s") * NC + lax.axis_index("c")
    np_w = jnp.where(wid < EXTRA, PER + 1, PER)
    p_lo = wid * PER + jnp.minimum(wid, EXTRA)
    id_lo = p_lo * PW
    id_hi = jnp.minimum((p_lo + np_w) * PW, NENT)
    lane = lax.iota(jnp.int32, L)

    sm[0] = jnp.int32(0)
    sm[1] = jnp.int32(0)
    sm[2] = jnp.int32(-1)
    sm[3] = jnp.int32(0)
    sm[4] = jnp.int32(0)

    pltpu.sync_copy(head_hbm, ids_v.at[pl.ds(0, B)])
    pltpu.sync_copy(tail_hbm, ids_v.at[pl.ds(B, B)])

    def prefilter(j, off):
        v = ids_v[pl.ds(j * L, L)]
        m = (v >= id_lo) & (v < id_hi)
        cs = plsc.cumsum(jnp.where(m, 1, 0))
        dest = off + cs - 1
        plsc.store_scatter(ids_v, [dest], v, mask=m)
        plsc.store_scatter(pos_v, [dest], j * L + lane, mask=m)
        return off + cs[15]

    cnt_total = lax.fori_loop(0, 2 * B // L, prefilter, jnp.int32(0))
    nck = (cnt_total + L - 1) // L

    def drain_sc():
        pltpu.make_async_copy(
            stage_v.at[0], gout_hbm.at[posb_v.at[0]], sem_sc).wait()

    def fire_e(p, b, sem):
        pg = p_lo + p

        @pl.when(pg < NP - 1)
        def _():
            st = pl.multiple_of(pg * PW, 128)
            pltpu.async_copy(entT_hbm.at[:, pl.ds(st, PW)], e_v.at[b], sem)

        @pl.when(pg == NP - 1)
        def _():
            for q in range(PW // 128):
                pltpu.async_copy(
                    tb_hbm, e_v.at[b, :, pl.ds(q * 128, 128)], sem)

    def drain_e(sem):
        pltpu.make_async_copy(
            entT_hbm.at[:, pl.ds(0, PW)], e_v.at[0], sem).wait()

    def compact_sub(s):
        slo = (p_lo + (s * np_w) // SUBS) * PW
        shi = jnp.minimum((p_lo + ((s + 1) * np_w) // SUBS) * PW, NENT)

        def cchunk(k, off):
            v = ids_v[pl.ds(k * L, L)]
            valid = (k * L + lane) < cnt_total
            m = valid & (v >= slo) & (v < shi)
            cs = plsc.cumsum(jnp.where(m, 1, 0))
            dest = jnp.minimum(off + cs - 1, CAP - 1)
            plsc.store_scatter(sids_v, [dest], v, mask=m)
            plsc.store_scatter(spos_v, [dest], pos_v[pl.ds(k * L, L)], mask=m)
            return off + cs[15]

        return lax.fori_loop(0, nck, cchunk, jnp.int32(0))

    def process(p, b):
        def drp(i, c):
            drain_sc()
            return c

        lax.fori_loop(0, sm[1], drp, jnp.int32(0))
        sm[1] = jnp.int32(0)

        s = (SUBS * (p + 1) - 1) // np_w

        @pl.when(s != sm[2])
        def _():
            sm[3] = compact_sub(s)
            sm[2] = s

        pg = p_lo + p
        rlo = pg * PW
        rhi = jnp.minimum(rlo + PW, NENT)
        start = jnp.where(pg == NP - 1, NENT - 128, rlo)
        csub = jnp.minimum(sm[3], CAP)

        def pchunk(k, off):
            v = sids_v[pl.ds(k * L, L)]
            valid = (k * L + lane) < csub
            m = valid & (v >= rlo) & (v < rhi)
            cs = plsc.cumsum(jnp.where(m, 1, 0))
            dest = jnp.minimum(off + cs - 1, 1024 - 1)
            plsc.store_scatter(ploc_v, [dest], v - start, mask=m)
            plsc.store_scatter(prow_v, [dest], spos_v[pl.ds(k * L, L)], mask=m)
            return off + cs[15]

        pcnt = lax.fori_loop(
            0, (csub + L - 1) // L, pchunk, jnp.int32(0))
        pcnt = jnp.minimum(pcnt, 1024)

        def strip(k, carry):
            mvalid = (k * L + lane) < pcnt
            lv = jnp.where(mvalid, ploc_v[pl.ds(k * L, L)], 0)
            rw = jnp.where(mvalid, prow_v[pl.ds(k * L, L)], DUMMY)
            ob = sm[1]

            @pl.when(ob >= 2)
            def _():
                drain_sc()

            r = sm[4]
            for c in range(D):
                cf = jnp.full((L,), c, jnp.int32)
                vals = plsc.load_gather(e_v.at[b], [cf, lv])
                plsc.store_scatter(stage_v.at[r], [lane, cf], vals)
            posb_v[r, pl.ds(0, L)] = rw
            pltpu.async_copy(stage_v.at[r], gout_hbm.at[posb_v.at[r]], sem_sc)
            sm[4] = lax.rem(r + 1, 2)
            sm[1] = jnp.where(ob >= 2, ob, ob + 1)
            return carry

        lax.fori_loop(0, (pcnt + L - 1) // L, strip, jnp.int32(0))

    fire_e(jnp.int32(0), 0, sem_e0)

    def piece(p, carry):
        par = lax.rem(p, 2)

        @pl.when((p + 1 < np_w) & (par == 0))
        def _():
            fire_e(p + 1, 1, sem_e1)

        @pl.when((p + 1 < np_w) & (par == 1))
        def _():
            fire_e(p + 1, 0, sem_e0)

        @pl.when(par == 0)
        def _():
            drain_e(sem_e0)

        @pl.when(par == 1)
        def _():
            drain_e(sem_e1)

        process(p, par)
        return carry

    lax.fori_loop(0, np_w, piece, jnp.int32(0))

    def dr_final(i, c):
        drain_sc()
        return c

    lax.fori_loop(0, sm[1], dr_final, jnp.int32(0))


CH2 = 64


@functools.partial(
    pl.kernel,
    mesh=_mesh,
    out_type=jax.ShapeDtypeStruct((B,), jnp.float32),
    compiler_params=_params,
    scratch_types=[
        pltpu.VMEM((BPW,), jnp.int32),
        pltpu.VMEM((D, NREL), jnp.float32),
        pltpu.VMEM((2, CH2, 128), jnp.float32),
        pltpu.VMEM((2, CH2, 128), jnp.float32),
        pltpu.VMEM((BPW,), jnp.float32),
        pltpu.SemaphoreType.DMA,
        pltpu.SemaphoreType.DMA,
        pltpu.SemaphoreType.DMA,
    ],
)
def _score_phase(rel_hbm, gout_hbm, relT_hbm, out_hbm,
                 ri_v, rel_v, h_v, t_v, o_v, sem_r, sem0, sem1):
    wid = lax.axis_index("s") * NC + lax.axis_index("c")
    base = wid * BPW
    lane = lax.iota(jnp.int32, L)

    pltpu.sync_copy(rel_hbm.at[pl.ds(base, BPW)], ri_v)
    rel_cp = pltpu.async_copy(relT_hbm, rel_v, sem_r)

    def fire(q, b, sem):
        r0 = base + q * CH2
        pltpu.async_copy(gout_hbm.at[pl.ds(r0, CH2)], h_v.at[b], sem)
        pltpu.async_copy(gout_hbm.at[pl.ds(B + r0, CH2)], t_v.at[b], sem)

    def drain(sem):
        for _ in range(2):
            pltpu.make_async_copy(
                gout_hbm.at[pl.ds(0, CH2)], h_v.at[0], sem).wait()

    def compute(q, b):
        def group(g, carry):
            j0 = g * L
            ridx = ri_v[pl.ds(q * CH2 + j0, L)]
            rows = j0 + lane
            acc = jnp.zeros((L,), jnp.float32)
            for c in range(D):
                cf = jnp.full((L,), c, jnp.int32)
                h = plsc.load_gather(h_v.at[b], [rows, cf])
                t = plsc.load_gather(t_v.at[b], [rows, cf])
                r = plsc.load_gather(rel_v, [cf, ridx])
                acc = acc + jnp.abs(h + r - t)
            o_v[pl.ds(q * CH2 + j0, L)] = acc
            return carry

        lax.fori_loop(0, CH2 // L, group, jnp.int32(0))

    NQ = BPW // CH2
    fire(jnp.int32(0), 0, sem0)
    rel_cp.wait()

    def pairq(qp, carry):
        q0 = qp * 2
        q1 = q0 + 1
        fire(q1, 1, sem1)
        drain(sem0)
        compute(q0, 0)

        @pl.when(q0 + 2 < NQ)
        def _():
            fire(q0 + 2, 0, sem0)

        drain(sem1)
        compute(q1, 1)
        return carry

    lax.fori_loop(0, NQ // 2, pairq, jnp.int32(0))
    pltpu.sync_copy(o_v, out_hbm.at[pl.ds(base, BPW)])


def kernel(head, relation, tail, entity_table, relation_table):
    tail_block = entity_table[NENT - 128:].T
    gout = _gather_phase(head, tail, entity_table.T, tail_block)
    return _score_phase(relation, gout, relation_table.T)


# speedup vs baseline: 3.5865x; 1.4336x over previous
import functools

import jax
import jax.numpy as jnp
from jax import lax
from jax.experimental import pallas as pl
from jax.experimental.pallas import tpu as pltpu
from jax.experimental.pallas import tpu_sc as plsc

B = 16384
D = 64
NENT = 1000000
NREL = 1000
L = 16
CH = 128

_info = plsc.get_sparse_core_info()
NC, NS = _info.num_cores, _info.num_subcores
NW = NC * NS
BPW = B // NW
HALF = BPW // 2
NCH = HALF // CH
NGRP = HALF // L

_mesh = plsc.VectorSubcoreMesh(core_axis_name="c", subcore_axis_name="s")


@functools.partial(
    pl.kernel,
    mesh=_mesh,
    out_type=jax.ShapeDtypeStruct((B,), jnp.float32),
    compiler_params=pltpu.CompilerParams(needs_layout_passes=False),
    scratch_types=[
        pltpu.VMEM((BPW // CH, CH), jnp.int32),
        pltpu.VMEM((BPW // CH, CH), jnp.int32),
        pltpu.VMEM((BPW // CH, CH), jnp.int32),
        pltpu.VMEM((BPW // CH, CH), jnp.int32),
        pltpu.VMEM((BPW // CH, CH), jnp.int32),
        pltpu.VMEM((BPW // CH, CH), jnp.int32),
        pltpu.VMEM((HALF, 2 * D), jnp.float32),
        pltpu.VMEM((HALF, 2 * D), jnp.float32),
        pltpu.VMEM((HALF, 2 * D), jnp.float32),
        pltpu.VMEM((BPW,), jnp.float32),
        pltpu.SemaphoreType.DMA,
    ],
)
def _transe_sc(head_hbm, rel_hbm, tail_hbm, ent2_hbm, rel2_hbm, out_hbm,
               hi_v, ti_v, ri_v, hp_v, tp_v, rp_v, h_v, t_v, r_v, o_v, sem):
    wid = lax.axis_index("s") * NC + lax.axis_index("c")
    base = wid * BPW
    crow = wid * (BPW // CH)

    pltpu.sync_copy(head_hbm.at[pl.ds(crow, BPW // CH)], hi_v)
    pltpu.sync_copy(tail_hbm.at[pl.ds(crow, BPW // CH)], ti_v)
    pltpu.sync_copy(rel_hbm.at[pl.ds(crow, BPW // CH)], ri_v)

    def shift_rows(j, carry):
        hp_v[j // 8, pl.ds((j % 8) * L, L)] = (
            hi_v[j // 8, pl.ds((j % 8) * L, L)] >> 1)
        tp_v[j // 8, pl.ds((j % 8) * L, L)] = (
            ti_v[j // 8, pl.ds((j % 8) * L, L)] >> 1)
        rp_v[j // 8, pl.ds((j % 8) * L, L)] = (
            ri_v[j // 8, pl.ds((j % 8) * L, L)] >> 1)
        return carry

    for j in range(BPW // L):
        shift_rows(j, 0)

    lane = lax.iota(jnp.int32, L)

    for half in range(2):
        off = half * HALF

        copies = []
        for c in range(NCH):
            crow_l = half * NCH + c
            dst = pl.ds(c * CH, CH)
            copies.append(pltpu.async_copy(
                ent2_hbm.at[hp_v.at[crow_l]], h_v.at[dst], sem))
            copies.append(pltpu.async_copy(
                ent2_hbm.at[tp_v.at[crow_l]], t_v.at[dst], sem))
            copies.append(pltpu.async_copy(
                rel2_hbm.at[rp_v.at[crow_l]], r_v.at[dst], sem))
        for cp in copies:
            cp.wait()

        def group_body(g, carry):
            j0 = g * L
            crow_l = half * NCH
            hraw = hi_v[crow_l + g // 8, pl.ds((g % 8) * L, L)]
            traw = ti_v[crow_l + g // 8, pl.ds((g % 8) * L, L)]
            rraw = ri_v[crow_l + g // 8, pl.ds((g % 8) * L, L)]
            hoff = (hraw & 1) * D
            toff = (traw & 1) * D
            roff = (rraw & 1) * D
            rows = j0 + lane
            acc = jnp.zeros((L,), jnp.float32)
            for c in range(D):
                h = plsc.load_gather(h_v, [rows, hoff + c])
                t = plsc.load_gather(t_v, [rows, toff + c])
                r = plsc.load_gather(r_v, [rows, roff + c])
                acc = acc + jnp.abs(h + r - t)
            o_v[pl.ds(off + j0, L)] = acc
            return carry

        lax.fori_loop(0, NGRP, group_body, jnp.int32(0))

    pltpu.sync_copy(o_v, out_hbm.at[pl.ds(base, BPW)])


def kernel(head, relation, tail, entity_table, relation_table):
    head2 = head.reshape(B // CH, CH)
    rel2 = relation.reshape(B // CH, CH)
    tail2 = tail.reshape(B // CH, CH)
    ent_pairs = entity_table.reshape(NENT // 2, 2 * D)
    rel_pairs = relation_table.reshape(NREL // 2, 2 * D)
    return _transe_sc(head2, rel2, tail2, ent_pairs, rel_pairs)


# speedup vs baseline: 3.8185x; 1.0647x over previous
import functools

import jax
import jax.numpy as jnp
from jax import lax
from jax.experimental import pallas as pl
from jax.experimental.pallas import tpu as pltpu
from jax.experimental.pallas import tpu_sc as plsc

B = 16384
D = 64
CH = 128
L = 16

_info = plsc.get_sparse_core_info()
NC, NS = _info.num_cores, _info.num_subcores
NW = NC * NS
BPW = B // NW
NCH = BPW // CH
NGRP = BPW // L

_mesh = plsc.VectorSubcoreMesh(core_axis_name="c", subcore_axis_name="s")


@functools.partial(
    pl.kernel,
    mesh=_mesh,
    out_type=jax.ShapeDtypeStruct((B,), jnp.float32),
    compiler_params=pltpu.CompilerParams(
        needs_layout_passes=False, use_tc_tiling_on_sc=False),
    scratch_types=[
        pltpu.VMEM((NCH, CH), jnp.int32),
        pltpu.VMEM((NCH, CH), jnp.int32),
        pltpu.VMEM((NCH, CH), jnp.int32),
        pltpu.VMEM((BPW, D), jnp.float32),
        pltpu.VMEM((BPW, D), jnp.float32),
        pltpu.VMEM((BPW, D), jnp.float32),
        pltpu.VMEM((BPW,), jnp.float32),
        pltpu.SemaphoreType.DMA,
    ],
)
def _transe_sc(head_hbm, rel_hbm, tail_hbm, ent_hbm, relt_hbm, out_hbm,
               hi_v, ri_v, ti_v, h_v, r_v, t_v, o_v, sem):
    wid = lax.axis_index("s") * NC + lax.axis_index("c")
    crow = wid * NCH

    pltpu.sync_copy(head_hbm.at[pl.ds(crow, NCH)], hi_v)
    pltpu.sync_copy(rel_hbm.at[pl.ds(crow, NCH)], ri_v)
    pltpu.sync_copy(tail_hbm.at[pl.ds(crow, NCH)], ti_v)

    copies = []
    for c in range(NCH):
        dst = pl.ds(c * CH, CH)
        copies.append(pltpu.async_copy(ent_hbm.at[hi_v.at[c]], h_v.at[dst], sem))
        copies.append(pltpu.async_copy(relt_hbm.at[ri_v.at[c]], r_v.at[dst], sem))
        copies.append(pltpu.async_copy(ent_hbm.at[ti_v.at[c]], t_v.at[dst], sem))
    for cp in copies:
        cp.wait()

    lane = lax.iota(jnp.int32, L)

    def group_body(g, carry):
        row0 = g * L
        out_vec = jnp.zeros((L,), jnp.float32)
        for i in range(L):
            row = row0 + i
            acc = jnp.zeros((L,), jnp.float32)
            for k in range(D // L):
                cs = pl.ds(k * L, L)
                h = h_v[row, cs]
                r = r_v[row, cs]
                t = t_v[row, cs]
                acc = acc + jnp.abs(h + r - t)
            out_vec = jnp.where(lane == i, jnp.sum(acc), out_vec)
        o_v[pl.ds(row0, L)] = out_vec
        return carry

    lax.fori_loop(0, NGRP, group_body, jnp.int32(0))

    pltpu.sync_copy(o_v, out_hbm.at[pl.ds(wid * BPW, BPW)])


def kernel(head, relation, tail, entity_table, relation_table):
    head2 = head.reshape(B // CH, CH)
    rel2 = relation.reshape(B // CH, CH)
    tail2 = tail.reshape(B // CH, CH)
    return _transe_sc(head2, rel2, tail2, entity_table, relation_table)
